# baseline (speedup 1.0000x reference)
import jax
import jax.numpy as jnp
from jax import lax
from jax.experimental import pallas as pl
from jax.experimental.pallas import tpu as pltpu
from jax.experimental.pallas import tpu_sc as plsc

B, L, H = 1024, 200, 64
MAXLEN, TYPE_VOCAB = 200, 2
N = B * L
EPS = 1e-5

_info = plsc.get_sparse_core_info()
NC, NS = _info.num_cores, _info.num_subcores
NW = NC * NS
PER_W = N // NW
C = 640
NCHUNK = PER_W // C


def _rsqrt(x):
    i = plsc.bitcast(x, jnp.int32)
    i = jnp.int32(0x5F3759DF) - (i >> 1)
    y = plsc.bitcast(i, jnp.float32)
    for _ in range(2):
        y = y * (1.5 - 0.5 * x * y * y)
    return y


def _body(x_hbm, pid_hbm, sid_hbm, word_hbm, posw_hbm, sentw_hbm, gam_hbm,
          bet_hbm, out_hbm,
          xidx0, xidx1, pidx0, pidx1, sidx0, sidx1, cidx,
          rows0, rows1, posw, sentw, combo, gamv, betv,
          gsem0, gsem1, osem0, osem1):
    wid = lax.axis_index("s") * NC + lax.axis_index("c")
    wbase = wid * PER_W

    xidx = (xidx0, xidx1)
    pidx = (pidx0, pidx1)
    sidx = (sidx0, sidx1)
    rows = (rows0, rows1)
    gsem = (gsem0, gsem1)
    osem = (osem0, osem1)

    pltpu.sync_copy(posw_hbm, posw)
    pltpu.sync_copy(sentw_hbm, sentw)
    pltpu.sync_copy(gam_hbm, gamv)
    pltpu.sync_copy(bet_hbm, betv)

    iota = lax.iota(jnp.int32, 16)
    kio = [iota + (k * 16) for k in range(4)]

    s0 = [sentw[0, pl.ds(k * 16, 16)] for k in range(4)]
    s1 = [sentw[1, pl.ds(k * 16, 16)] for k in range(4)]

    @plsc.parallel_loop(0, MAXLEN)
    def _build(p):
        for k in range(4):
            pr = posw[p, pl.ds(k * 16, 16)]
            combo[2 * p, pl.ds(k * 16, 16)] = pr + s0[k]
            combo[2 * p + 1, pl.ds(k * 16, 16)] = pr + s1[k]

    gk = [gamv[pl.ds(k * 16, 16)] for k in range(4)]
    bk = [betv[pl.ds(k * 16, 16)] for k in range(4)]

    pltpu.sync_copy(x_hbm.at[pl.ds(wbase, C)], xidx[0])
    pltpu.sync_copy(pid_hbm.at[pl.ds(wbase, C)], pidx[0])
    pltpu.sync_copy(sid_hbm.at[pl.ds(wbase, C)], sidx[0])
    pltpu.async_copy(word_hbm.at[xidx[0]], rows[0], gsem[0])

    def pair_body(j, carry):
        for b in range(2):
            ci = 2 * j + b
            nb = 1 - b
            base = wbase + ci * C

            @pl.when(ci + 1 < NCHUNK)
            def _prefetch():
                nbase = base + C
                pltpu.sync_copy(x_hbm.at[pl.ds(nbase, C)], xidx[nb])
                pltpu.sync_copy(pid_hbm.at[pl.ds(nbase, C)], pidx[nb])
                pltpu.sync_copy(sid_hbm.at[pl.ds(nbase, C)], sidx[nb])

                @pl.when(ci >= 1)
                def _drain_out():
                    pltpu.make_async_copy(
                        rows[nb], out_hbm.at[pl.ds(wbase, C)], osem[nb]
                    ).wait()

                pltpu.async_copy(word_hbm.at[xidx[nb]], rows[nb], gsem[nb])

            pltpu.make_async_copy(
                word_hbm.at[xidx[b]], rows[b], gsem[b]).wait()

            @plsc.parallel_loop(0, C // 16)
            def _mkcidx(g):
                t0 = g * 16
                cidx[pl.ds(t0, 16)] = (pidx[b][pl.ds(t0, 16)] * 2
                                       + sidx[b][pl.ds(t0, 16)])

            rb = rows[b]

            @plsc.parallel_loop(0, C, unroll=8)
            def _tok(t):
                tsplat = jnp.full((16,), t, jnp.int32)
                csplat = plsc.load_gather(cidx, [tsplat])
                w = [rb[t, pl.ds(k * 16, 16)] for k in range(4)]
                cv = [plsc.load_gather(combo, [csplat, kio[k]])
                      for k in range(4)]
                v = [w[k] + cv[k] for k in range(4)]
                sq = [v[k] * v[k] for k in range(4)]
                tot = jnp.sum((v[0] + v[1]) + (v[2] + v[3]))
                totq = jnp.sum((sq[0] + sq[1]) + (sq[2] + sq[3]))
                mean = jnp.full((16,), tot, jnp.float32) * (1.0 / H)
                ex2 = jnp.full((16,), totq, jnp.float32) * (1.0 / H)
                r = _rsqrt(ex2 - mean * mean + EPS)
                m2 = -(mean * r)
                for k in range(4):
                    rb[t, pl.ds(k * 16, 16)] = (v[k] * r + m2) * gk[k] + bk[k]

            pltpu.async_copy(rb, out_hbm.at[pl.ds(base, C)], osem[b])
        return carry

    lax.fori_loop(0, NCHUNK // 2, pair_body, 0, unroll=False)

    for b in range(2):
        pltpu.make_async_copy(
            rows[b], out_hbm.at[pl.ds(wbase, C)], osem[b]).wait()


def kernel(x, pos_ids, sent_ids, word_W, pos_W, sent_W, gamma, beta):
    xf = x.reshape(N).astype(jnp.int32)
    pf = pos_ids.reshape(N).astype(jnp.int32)
    sf = sent_ids.reshape(N).astype(jnp.int32)
    mesh = plsc.VectorSubcoreMesh(core_axis_name="c", subcore_axis_name="s")
    f = pl.kernel(
        _body,
        out_type=jax.ShapeDtypeStruct((N, H), jnp.float32),
        mesh=mesh,
        compiler_params=pltpu.CompilerParams(needs_layout_passes=False,
                                             use_tc_tiling_on_sc=False),
        scratch_types=[
            pltpu.VMEM((C,), jnp.int32),
            pltpu.VMEM((C,), jnp.int32),
            pltpu.VMEM((C,), jnp.int32),
            pltpu.VMEM((C,), jnp.int32),
            pltpu.VMEM((C,), jnp.int32),
            pltpu.VMEM((C,), jnp.int32),
            pltpu.VMEM((C,), jnp.int32),
            pltpu.VMEM((C, H), jnp.float32),
            pltpu.VMEM((C, H), jnp.float32),
            pltpu.VMEM((MAXLEN, H), jnp.float32),
            pltpu.VMEM((TYPE_VOCAB, H), jnp.float32),
            pltpu.VMEM((2 * MAXLEN, H), jnp.float32),
            pltpu.VMEM((H,), jnp.float32),
            pltpu.VMEM((H,), jnp.float32),
            pltpu.SemaphoreType.DMA,
            pltpu.SemaphoreType.DMA,
            pltpu.SemaphoreType.DMA,
            pltpu.SemaphoreType.DMA,
        ],
    )
    out = f(xf, pf, sf, word_W.astype(jnp.float32), pos_W.astype(jnp.float32),
            sent_W.astype(jnp.float32), gamma.astype(jnp.float32),
            beta.astype(jnp.float32))
    return out.reshape(B, L, H)


# speedup vs baseline: 1.0686x; 1.0686x over previous
import jax
import jax.numpy as jnp
from jax import lax
from jax.experimental import pallas as pl
from jax.experimental.pallas import tpu as pltpu
from jax.experimental.pallas import tpu_sc as plsc

B, L, H = 1024, 200, 64
MAXLEN, TYPE_VOCAB = 200, 2
N = B * L
EPS = 1e-5

_info = plsc.get_sparse_core_info()
NC, NS = _info.num_cores, _info.num_subcores
NW = NC * NS
PER_W = N // NW
C = 320
NCHUNK = PER_W // C


def _rsqrt(x):
    i = plsc.bitcast(x, jnp.int32)
    i = jnp.int32(0x5F3759DF) - (i >> 1)
    y = plsc.bitcast(i, jnp.float32)
    for _ in range(2):
        y = y * (1.5 - 0.5 * x * y * y)
    return y


def _body(x_hbm, pid_hbm, sid_hbm, word_hbm, posw_hbm, sentw_hbm, gam_hbm,
          bet_hbm, out_hbm,
          xidx0, xidx1, pidx0, pidx1, sidx0, sidx1, cidx,
          rows0, rows1, posw, sentw, combo, gamv, betv,
          gsem0, gsem1, osem0, osem1):
    wid = lax.axis_index("s") * NC + lax.axis_index("c")
    wbase = wid * PER_W

    xidx = (xidx0, xidx1)
    pidx = (pidx0, pidx1)
    sidx = (sidx0, sidx1)
    rows = (rows0, rows1)
    gsem = (gsem0, gsem1)
    osem = (osem0, osem1)

    pltpu.sync_copy(posw_hbm, posw)
    pltpu.sync_copy(sentw_hbm, sentw)
    pltpu.sync_copy(gam_hbm, gamv)
    pltpu.sync_copy(bet_hbm, betv)

    iota = lax.iota(jnp.int32, 16)
    kio = [iota + (k * 16) for k in range(4)]

    s0 = [sentw[0, pl.ds(k * 16, 16)] for k in range(4)]
    s1 = [sentw[1, pl.ds(k * 16, 16)] for k in range(4)]

    @plsc.parallel_loop(0, MAXLEN)
    def _build(p):
        for k in range(4):
            pr = posw[p, pl.ds(k * 16, 16)]
            combo[2 * p, pl.ds(k * 16, 16)] = pr + s0[k]
            combo[2 * p + 1, pl.ds(k * 16, 16)] = pr + s1[k]

    gk = [gamv[pl.ds(k * 16, 16)] for k in range(4)]
    bk = [betv[pl.ds(k * 16, 16)] for k in range(4)]

    pltpu.sync_copy(x_hbm.at[pl.ds(wbase, C)], xidx[0])
    pltpu.sync_copy(pid_hbm.at[pl.ds(wbase, C)], pidx[0])
    pltpu.sync_copy(sid_hbm.at[pl.ds(wbase, C)], sidx[0])
    pltpu.async_copy(word_hbm.at[xidx[0]], rows[0], gsem[0])

    def pair_body(j, carry):
        for b in range(2):
            ci = 2 * j + b
            nb = 1 - b
            base = wbase + ci * C

            @pl.when(ci + 1 < NCHUNK)
            def _prefetch():
                nbase = base + C
                pltpu.sync_copy(x_hbm.at[pl.ds(nbase, C)], xidx[nb])
                pltpu.sync_copy(pid_hbm.at[pl.ds(nbase, C)], pidx[nb])
                pltpu.sync_copy(sid_hbm.at[pl.ds(nbase, C)], sidx[nb])

                @pl.when(ci >= 1)
                def _drain_out():
                    pltpu.make_async_copy(
                        rows[nb], out_hbm.at[pl.ds(wbase, C)], osem[nb]
                    ).wait()

                pltpu.async_copy(word_hbm.at[xidx[nb]], rows[nb], gsem[nb])

            pltpu.make_async_copy(
                word_hbm.at[xidx[b]], rows[b], gsem[b]).wait()

            @plsc.parallel_loop(0, C // 16)
            def _mkcidx(g):
                t0 = g * 16
                cidx[pl.ds(t0, 16)] = (pidx[b][pl.ds(t0, 16)] * 2
                                       + sidx[b][pl.ds(t0, 16)])

            rb = rows[b]

            @plsc.parallel_loop(0, C, unroll=4)
            def _tok(t):
                tsplat = jnp.full((16,), t, jnp.int32)
                csplat = plsc.load_gather(cidx, [tsplat])
                w = [rb[t, pl.ds(k * 16, 16)] for k in range(4)]
                cv = [plsc.load_gather(combo, [csplat, kio[k]])
                      for k in range(4)]
                v = [w[k] + cv[k] for k in range(4)]
                sq = [v[k] * v[k] for k in range(4)]
                tot = jnp.sum((v[0] + v[1]) + (v[2] + v[3]))
                totq = jnp.sum((sq[0] + sq[1]) + (sq[2] + sq[3]))
                mean = jnp.full((16,), tot, jnp.float32) * (1.0 / H)
                ex2 = jnp.full((16,), totq, jnp.float32) * (1.0 / H)
                r = _rsqrt(ex2 - mean * mean + EPS)
                m2 = -(mean * r)
                for k in range(4):
                    rb[t, pl.ds(k * 16, 16)] = (v[k] * r + m2) * gk[k] + bk[k]

            pltpu.async_copy(rb, out_hbm.at[pl.ds(base, C)], osem[b])
        return carry

    lax.fori_loop(0, NCHUNK // 2, pair_body, 0, unroll=False)

    for b in range(2):
        pltpu.make_async_copy(
            rows[b], out_hbm.at[pl.ds(wbase, C)], osem[b]).wait()


def kernel(x, pos_ids, sent_ids, word_W, pos_W, sent_W, gamma, beta):
    xf = x.reshape(N).astype(jnp.int32)
    pf = pos_ids.reshape(N).astype(jnp.int32)
    sf = sent_ids.reshape(N).astype(jnp.int32)
    mesh = plsc.VectorSubcoreMesh(core_axis_name="c", subcore_axis_name="s")
    f = pl.kernel(
        _body,
        out_type=jax.ShapeDtypeStruct((N, H), jnp.float32),
        mesh=mesh,
        compiler_params=pltpu.CompilerParams(needs_layout_passes=False,
                                             use_tc_tiling_on_sc=False),
        scratch_types=[
            pltpu.VMEM((C,), jnp.int32),
            pltpu.VMEM((C,), jnp.int32),
            pltpu.VMEM((C,), jnp.int32),
            pltpu.VMEM((C,), jnp.int32),
            pltpu.VMEM((C,), jnp.int32),
            pltpu.VMEM((C,), jnp.int32),
            pltpu.VMEM((C,), jnp.int32),
            pltpu.VMEM((C, H), jnp.float32),
            pltpu.VMEM((C, H), jnp.float32),
            pltpu.VMEM((MAXLEN, H), jnp.float32),
            pltpu.VMEM((TYPE_VOCAB, H), jnp.float32),
            pltpu.VMEM((2 * MAXLEN, H), jnp.float32),
            pltpu.VMEM((H,), jnp.float32),
            pltpu.VMEM((H,), jnp.float32),
            pltpu.SemaphoreType.DMA,
            pltpu.SemaphoreType.DMA,
            pltpu.SemaphoreType.DMA,
            pltpu.SemaphoreType.DMA,
        ],
    )
    out = f(xf, pf, sf, word_W.astype(jnp.float32), pos_W.astype(jnp.float32),
            sent_W.astype(jnp.float32), gamma.astype(jnp.float32),
            beta.astype(jnp.float32))
    return out.reshape(B, L, H)
